# baseline (speedup 1.0000x reference)
import functools

import jax
import jax.numpy as jnp
from jax.experimental import pallas as pl
from jax.experimental.pallas import tpu as pltpu


def _half(y, x):
    shape3 = x.shape
    y3 = jax.lax.broadcast_in_dim(y, shape3, (0, 1))
    mask3 = y3 != -1
    safe3 = jnp.where(mask3, y3, 0)
    iota3 = jax.lax.broadcasted_iota(jnp.int32, shape3, 2)
    g = jnp.sum(jnp.where(iota3 == safe3, x, 0.0), axis=-1)
    lse = jnp.log(jnp.sum(jnp.exp(x), axis=-1))
    loss = jnp.where(y != -1, lse - g, 0.0)
    return jnp.sum(loss, axis=-1, keepdims=True)


def _body(ya_ref, xa_ref, yb_ref, xb_ref, oa_ref, ob_ref):
    oa_ref[0] = _half(ya_ref[...], xa_ref[...])
    ob_ref[0] = _half(yb_ref[...], xb_ref[...])


def kernel(y_true, y_pred):
    b, a, c = y_pred.shape
    bbb = 64
    half = b // (2 * bbb)

    yi = y_true.astype(jnp.int32)
    oa, ob = pl.pallas_call(
        _body,
        grid=(half,),
        in_specs=[
            pl.BlockSpec((bbb, a), lambda i: (i, 0)),
            pl.BlockSpec((bbb, a, c), lambda i: (i, 0, 0)),
            pl.BlockSpec((bbb, a), lambda i: (i + half, 0)),
            pl.BlockSpec((bbb, a, c), lambda i: (i + half, 0, 0)),
        ],
        out_specs=[
            pl.BlockSpec((1, bbb, 1), lambda i: (i, 0, 0)),
            pl.BlockSpec((1, bbb, 1), lambda i: (i, 0, 0)),
        ],
        out_shape=[
            jax.ShapeDtypeStruct((half, bbb, 1), jnp.float32),
            jax.ShapeDtypeStruct((half, bbb, 1), jnp.float32),
        ],
    )(yi, y_pred, yi, y_pred)
    return jnp.concatenate([oa.reshape(b // 2), ob.reshape(b // 2)])
